# baseline (speedup 1.0000x reference)
import functools

import jax
import jax.numpy as jnp
from jax import lax
from jax.experimental import pallas as pl
from jax.experimental.pallas import tpu as pltpu
from jax.experimental.pallas import tpu_sc as plsc

F32 = jnp.float32
I32 = jnp.int32

_N_FULL = 100000
_N_SUB = 10000
_E = 320000
_D = 128
_H = 256
_C = 40

_NC = 2
_NS = 16
_NW = _NC * _NS

_NSP = 10240
_EP = 327680
_EPR = _EP // 128
_B = 128
_CB = 16

_RT1 = _NSP // _NW
_EB1 = _EP // _NW // _B
_EB2 = _EP // _NS // _B
_ZT = _NSP // _NS
_DW = 16
_A1R = 10112
_A1T = _A1R // _NS

_mesh = plsc.VectorSubcoreMesh(core_axis_name="c", subcore_axis_name="s")


def _zero_acc(buf, acc, s, rpt):
    @pl.loop(0, _B)
    def _z(r):
        for k in range(_D // 16):
            buf[r, pl.ds(k * 16, 16)] = jnp.zeros((16,), F32)

    base = s * rpt
    for off in range(0, rpt, _B):
        n = min(_B, rpt - off)
        pltpu.sync_copy(buf.at[pl.ds(0, n)], acc.at[pl.ds(base + off, n)])


def _agg_chunk(tab, idx_v, dst_v, buf_a, buf_b, acc, gsem_a, gsem_b):
    def _gather(j, buf, sem):
        return pltpu.async_copy(tab.at[idx_v.at[j]], buf, sem)

    def _wait(buf, sem):
        pltpu.make_async_copy(tab.at[idx_v.at[0]], buf, sem).wait()

    _gather(0, buf_a, gsem_a)

    @pl.loop(0, _CB // 2)
    def _agg(t):
        j0 = 2 * t
        j1 = j0 + 1
        _wait(buf_a, gsem_a)
        _gather(j1, buf_b, gsem_b)
        pltpu.sync_copy(buf_a, acc.at[dst_v.at[j0]], add=True)
        _wait(buf_b, gsem_b)
        _gather(jnp.minimum(j0 + 2, _CB - 1), buf_a, gsem_a)
        pltpu.sync_copy(buf_b, acc.at[dst_v.at[j1]], add=True)

    _wait(buf_a, gsem_a)


def _sc1a_body(feat_hbm, ns_hbm, labtab_hbm, dst_hbm,
               fs_hbm, fs2_hbm, lab_hbm, deg_hbm,
               ns_v, dst_v, buf_a, lab_v, acc2,
               gsem_a, gsem_b):
    c = lax.axis_index("c")
    s = lax.axis_index("s")
    wid = c * _NS + s

    _zero_acc(buf_a, acc2, s, _A1T)

    base = wid * _RT1
    pltpu.sync_copy(ns_hbm.at[pl.ds(base, _RT1)], ns_v)

    d0 = pltpu.async_copy(feat_hbm.at[ns_v.at[pl.ds(0, 128)]], buf_a, gsem_a)
    d0.wait()
    pltpu.sync_copy(buf_a, fs_hbm.at[pl.ds(base, 128)])
    pltpu.sync_copy(buf_a, fs2_hbm.at[pl.ds(base, 128)])
    d1 = pltpu.async_copy(feat_hbm.at[ns_v.at[pl.ds(128, 128)]], buf_a,
                          gsem_b)
    d1.wait()
    pltpu.sync_copy(buf_a, fs_hbm.at[pl.ds(base + 128, 128)])
    pltpu.sync_copy(buf_a, fs2_hbm.at[pl.ds(base + 128, 128)])
    d2 = pltpu.async_copy(feat_hbm.at[ns_v.at[pl.ds(256, 64)]],
                          buf_a.at[pl.ds(0, 64)], gsem_a)
    d2.wait()
    pltpu.sync_copy(buf_a.at[pl.ds(0, 64)], fs_hbm.at[pl.ds(base + 256, 64)])
    pltpu.sync_copy(buf_a.at[pl.ds(0, 64)], fs2_hbm.at[pl.ds(base + 256, 64)])

    l0 = pltpu.async_copy(labtab_hbm.at[ns_v.at[pl.ds(0, 128)]],
                          lab_v.at[pl.ds(0, 128)], gsem_a)
    l1 = pltpu.async_copy(labtab_hbm.at[ns_v.at[pl.ds(128, 128)]],
                          lab_v.at[pl.ds(128, 128)], gsem_b)
    l0.wait()
    l1.wait()
    l2 = pltpu.async_copy(labtab_hbm.at[ns_v.at[pl.ds(256, 64)]],
                          lab_v.at[pl.ds(256, 64)], gsem_a)
    l2.wait()
    pltpu.sync_copy(lab_v, lab_hbm.at[pl.ds(base, _RT1)])

    @pl.loop(0, _B)
    def _o(r):
        for k in range(_D // 16):
            buf_a[r, pl.ds(k * 16, 16)] = jnp.ones((16,), F32)

    plsc.subcore_barrier()

    erow = wid * _EB1
    for ch in range(_EB1 // _CB):
        pltpu.sync_copy(dst_hbm.at[pl.ds(erow + ch * _CB, _CB)], dst_v)

        @pl.loop(0, _CB)
        def _deg(j):
            pltpu.sync_copy(buf_a, acc2.at[dst_v.at[j]], add=True)

    plsc.subcore_barrier()
    pltpu.sync_copy(acc2.at[pl.ds(s * _A1T, _A1T)],
                    deg_hbm.at[pl.ds(c * _NSP + s * _A1T, _A1T)])


_sc1a = functools.partial(
    pl.kernel,
    out_type=[
        jax.ShapeDtypeStruct((_NSP, _D), F32),
        jax.ShapeDtypeStruct((_NSP, _D), F32),
        jax.ShapeDtypeStruct((_NSP,), I32),
        jax.ShapeDtypeStruct((2 * _NSP, _D), F32),
    ],
    mesh=_mesh,
    scratch_types=[
        pltpu.VMEM((_RT1,), I32),
        pltpu.VMEM((_CB, _B), I32),
        pltpu.VMEM((_B, _D), F32),
        pltpu.VMEM((_RT1,), I32),
        pltpu.VMEM_SHARED((_A1R, _D), F32),
        pltpu.SemaphoreType.DMA,
        pltpu.SemaphoreType.DMA,
    ],
)(_sc1a_body)


def _sc1b_body(fs_hbm, fs2_hbm, src_hbm, dst_hbm,
               agg1_hbm,
               idx_v, dst_v, buf_a, buf_b, acc,
               gsem_a, gsem_b):
    c = lax.axis_index("c")
    s = lax.axis_index("s")
    wid = c * _NS + s

    _zero_acc(buf_a, acc, s, _A1T)
    plsc.subcore_barrier()

    erow = wid * _EB1
    for ch in range(_EB1 // _CB):
        r0 = erow + ch * _CB
        pltpu.sync_copy(src_hbm.at[pl.ds(r0, _CB)], idx_v)
        pltpu.sync_copy(dst_hbm.at[pl.ds(r0, _CB)], dst_v)

        @pl.when(c == 0)
        def _():
            _agg_chunk(fs_hbm, idx_v, dst_v, buf_a, buf_b, acc,
                       gsem_a, gsem_b)

        @pl.when(c == 1)
        def _():
            _agg_chunk(fs2_hbm, idx_v, dst_v, buf_a, buf_b, acc,
                       gsem_a, gsem_b)

    plsc.subcore_barrier()
    pltpu.sync_copy(acc.at[pl.ds(s * _A1T, _A1T)],
                    agg1_hbm.at[pl.ds(c * _NSP + s * _A1T, _A1T)])


_sc1b = functools.partial(
    pl.kernel,
    out_type=jax.ShapeDtypeStruct((2 * _NSP, _D), F32),
    mesh=_mesh,
    scratch_types=[
        pltpu.VMEM((_CB, _B), I32),
        pltpu.VMEM((_CB, _B), I32),
        pltpu.VMEM((_B, _D), F32),
        pltpu.VMEM((_B, _D), F32),
        pltpu.VMEM_SHARED((_A1R, _D), F32),
        pltpu.SemaphoreType.DMA,
        pltpu.SemaphoreType.DMA,
    ],
)(_sc1b_body)


def _sc2_body(y2a_hbm, y2b_hbm, src_hbm, dst_hbm,
              agg2_hbm,
              idx_v, dst_v, buf_a, buf_b, acc, gsem_a, gsem_b):
    c = lax.axis_index("c")
    s = lax.axis_index("s")

    _zero_acc(buf_a, acc, s, _ZT)
    plsc.subcore_barrier()

    erow = s * _EB2
    for ch in range(_EB2 // _CB):
        r0 = erow + ch * _CB
        pltpu.sync_copy(src_hbm.at[pl.ds(r0, _CB)], idx_v)
        pltpu.sync_copy(dst_hbm.at[pl.ds(r0, _CB)], dst_v)

        @pl.when(c == 0)
        def _():
            _agg_chunk(y2a_hbm, idx_v, dst_v, buf_a, buf_b, acc,
                       gsem_a, gsem_b)

        @pl.when(c == 1)
        def _():
            _agg_chunk(y2b_hbm, idx_v, dst_v, buf_a, buf_b, acc,
                       gsem_a, gsem_b)

    plsc.subcore_barrier()
    pltpu.sync_copy(acc.at[pl.ds(s * _ZT, _ZT)],
                    agg2_hbm.at[pl.ds(c * _NSP + s * _ZT, _ZT)])


_sc2 = functools.partial(
    pl.kernel,
    out_type=jax.ShapeDtypeStruct((2 * _NSP, _D), F32),
    mesh=_mesh,
    scratch_types=[
        pltpu.VMEM((_CB, _B), I32),
        pltpu.VMEM((_CB, _B), I32),
        pltpu.VMEM((_B, _D), F32),
        pltpu.VMEM((_B, _D), F32),
        pltpu.VMEM_SHARED((_NSP, _D), F32),
        pltpu.SemaphoreType.DMA,
        pltpu.SemaphoreType.DMA,
    ],
)(_sc2_body)


_R = 512
_G = _NSP // _R


def _tc1_body(fs, agg, dgp, ss, w1s, b1s, sn, w1n, b1n, w2s, b2s, w2n,
              xs2_o, y2a_o, y2b_o):
    dg = dgp[0, :, :1] + dgp[1, :, :1]
    scale = 1.0 / jnp.maximum(dg, 1.0)
    w1se = jnp.dot(ss[...], w1s[...], preferred_element_type=F32)
    w1ne = jnp.dot(sn[...], w1n[...], preferred_element_type=F32)
    a1 = (agg[0] + agg[1]) * scale
    xs = jnp.dot(fs[...], w1se, preferred_element_type=F32) + b1s[...]
    xn = jnp.dot(a1, w1ne, preferred_element_type=F32) + b1n[...]
    hs = jnp.maximum(xs, 0.0)
    hn = jnp.maximum(xn, 0.0)
    w2s_ = w2s[...]
    w2n_ = w2n[...]
    xs2_o[...] = (jnp.dot(hs, w2s_[:_H], preferred_element_type=F32)
                  + jnp.dot(hn, w2s_[_H:], preferred_element_type=F32) + b2s[...])
    y2 = (jnp.dot(hs, w2n_[:_H], preferred_element_type=F32)
          + jnp.dot(hn, w2n_[_H:], preferred_element_type=F32))
    y2a_o[...] = y2[:, :_D]
    y2b_o[...] = y2[:, _D:]


def _tc1(fs, agg, dgp, ss, w1s, b1s, sn, w1n, b1n, w2s, b2s, w2n):
    full = lambda shp: pl.BlockSpec(shp, lambda i: tuple(0 for _ in shp))
    return pl.pallas_call(
        _tc1_body,
        grid=(_G,),
        in_specs=[
            pl.BlockSpec((_R, _D), lambda i: (i, 0)),
            pl.BlockSpec((2, _R, _D), lambda i: (0, i, 0)),
            pl.BlockSpec((2, _R, _D), lambda i: (0, i, 0)),
            full((_D, _D)), full((_D, _H)), full((1, _H)),
            full((_D, _D)), full((_D, _H)), full((1, _H)),
            full((2 * _H, _H)), full((1, _H)), full((2 * _H, _H)),
        ],
        out_specs=[
            pl.BlockSpec((_R, _H), lambda i: (i, 0)),
            pl.BlockSpec((_R, _D), lambda i: (i, 0)),
            pl.BlockSpec((_R, _D), lambda i: (i, 0)),
        ],
        out_shape=[
            jax.ShapeDtypeStruct((_NSP, _H), F32),
            jax.ShapeDtypeStruct((_NSP, _D), F32),
            jax.ShapeDtypeStruct((_NSP, _D), F32),
        ],
    )(fs, agg, dgp, ss, w1s, b1s, sn, w1n, b1n, w2s, b2s, w2n)


def _tc2_body(xs2, agg, dgp, b2n, wc, bc, labf, pred_o, lab_o):
    dg = dgp[0, :, :1] + dgp[1, :, :1]
    scale = 1.0 / jnp.maximum(dg, 1.0)
    xn2 = jnp.concatenate([agg[0], agg[1]], axis=1) * scale + b2n[...]
    h2 = jnp.concatenate([jnp.maximum(xs2[...], 0.0), jnp.maximum(xn2, 0.0)],
                         axis=1)
    ssum = jnp.sum(h2 * h2, axis=1, keepdims=True)
    emb = h2 / jnp.maximum(jnp.sqrt(ssum), 1e-12)
    pred_o[...] = jnp.dot(emb, wc[...], preferred_element_type=F32) + bc[...]
    cls = lax.broadcasted_iota(I32, (_R, _D), 1).astype(F32)
    lab_o[...] = (cls == labf[...]).astype(F32)


def _tc2(xs2, agg, dgp, b2n, wc, bc, labf):
    full = lambda shp: pl.BlockSpec(shp, lambda i: tuple(0 for _ in shp))
    return pl.pallas_call(
        _tc2_body,
        grid=(_G,),
        in_specs=[
            pl.BlockSpec((_R, _H), lambda i: (i, 0)),
            pl.BlockSpec((2, _R, _D), lambda i: (0, i, 0)),
            pl.BlockSpec((2, _R, _D), lambda i: (0, i, 0)),
            full((1, _H)),
            full((2 * _H, _D)), full((1, _D)),
            pl.BlockSpec((_R, 1), lambda i: (i, 0)),
        ],
        out_specs=[
            pl.BlockSpec((_R, _D), lambda i: (i, 0)),
            pl.BlockSpec((_R, _D), lambda i: (i, 0)),
        ],
        out_shape=[
            jax.ShapeDtypeStruct((_NSP, _D), F32),
            jax.ShapeDtypeStruct((_NSP, _D), F32),
        ],
    )(xs2, agg, dgp, b2n, wc, bc, labf)


def kernel(node_subgraph, edge_index, edge_vals, feat_full, label_full,
           label_full_cat, mask_self, mask_neigh, W1_self, b1_self, W1_neigh,
           b1_neigh, W2_self, b2_self, W2_neigh, b2_neigh, Wc, bc):
    ns_pad = jnp.concatenate(
        [node_subgraph, jnp.zeros((_NSP - _N_SUB,), I32)])
    src_pad = jnp.concatenate(
        [edge_index[1], jnp.zeros((_EP - _E,), I32)]).reshape(_EPR, _B)
    dst_pad = jnp.concatenate(
        [edge_index[0], jnp.full((_EP - _E,), _N_SUB, I32)]).reshape(_EPR, _B)

    iota_d = jnp.arange(_D, dtype=I32)
    s_self = jnp.pad((iota_d[:, None] == mask_self[None, :]).astype(F32),
                     ((0, 0), (0, _D - mask_self.shape[0])))
    s_neigh = jnp.pad((iota_d[:, None] == mask_neigh[None, :]).astype(F32),
                      ((0, 0), (0, _D - mask_neigh.shape[0])))
    w1s_p = jnp.pad(W1_self, ((0, _D - W1_self.shape[0]), (0, 0)))
    w1n_p = jnp.pad(W1_neigh, ((0, _D - W1_neigh.shape[0]), (0, 0)))
    wc_p = jnp.pad(Wc, ((0, 0), (0, _D - _C)))
    bc_p = jnp.pad(bc, (0, _D - _C)).reshape(1, _D)

    fs, fs2, labcat, deg = _sc1a(feat_full, ns_pad, label_full_cat, dst_pad)
    agg1 = _sc1b(fs, fs2, src_pad, dst_pad)
    dgp = deg.reshape(2, _NSP, _D)

    xs2, y2a, y2b = _tc1(
        fs, agg1.reshape(2, _NSP, _D), dgp,
        s_self, w1s_p, b1_self.reshape(1, _H),
        s_neigh, w1n_p, b1_neigh.reshape(1, _H),
        W2_self, b2_self.reshape(1, _H), W2_neigh)

    agg2 = _sc2(y2a, y2b, src_pad, dst_pad)

    pred_p, lab_p = _tc2(
        xs2, agg2.reshape(2, _NSP, _D), dgp,
        b2_neigh.reshape(1, _H), wc_p, bc_p,
        labcat.astype(F32).reshape(_NSP, 1))

    return (pred_p[:_N_SUB, :_C], lab_p[:_N_SUB, :_C], labcat[:_N_SUB])


# speedup vs baseline: 1.0430x; 1.0430x over previous
import functools

import jax
import jax.numpy as jnp
from jax import lax
from jax.experimental import pallas as pl
from jax.experimental.pallas import tpu as pltpu
from jax.experimental.pallas import tpu_sc as plsc

F32 = jnp.float32
I32 = jnp.int32

_N_FULL = 100000
_N_SUB = 10000
_E = 320000
_D = 128
_H = 256
_C = 40

_NC = 2
_NS = 16
_NW = _NC * _NS

_NSP = 10240
_EP = 327680
_EPR = _EP // 128
_B = 128
_CB = 16

_RT1 = _NSP // _NW
_EB1 = _EP // _NW // _B
_EB2 = _EP // _NS // _B
_ZT = _NSP // _NS
_DW = 16
_A1R = 10112
_A1T = _A1R // _NS

_mesh = plsc.VectorSubcoreMesh(core_axis_name="c", subcore_axis_name="s")


def _zero_acc(buf, acc, s, rpt):
    @pl.loop(0, _B)
    def _z(r):
        for k in range(_D // 16):
            buf[r, pl.ds(k * 16, 16)] = jnp.zeros((16,), F32)

    base = s * rpt
    for off in range(0, rpt, _B):
        n = min(_B, rpt - off)
        pltpu.sync_copy(buf.at[pl.ds(0, n)], acc.at[pl.ds(base + off, n)])


def _agg_chunk(tab, idx_v, dst_v, buf_a, buf_b, acc, gsem_a, gsem_b):
    def _gather(j, buf, sem):
        return pltpu.async_copy(tab.at[idx_v.at[j]], buf, sem)

    def _wait(buf, sem):
        pltpu.make_async_copy(tab.at[idx_v.at[0]], buf, sem).wait()

    _gather(0, buf_a, gsem_a)

    @pl.loop(0, _CB // 2)
    def _agg(t):
        j0 = 2 * t
        j1 = j0 + 1
        _wait(buf_a, gsem_a)
        _gather(j1, buf_b, gsem_b)
        pltpu.sync_copy(buf_a, acc.at[dst_v.at[j0]], add=True)
        _wait(buf_b, gsem_b)
        _gather(jnp.minimum(j0 + 2, _CB - 1), buf_a, gsem_a)
        pltpu.sync_copy(buf_b, acc.at[dst_v.at[j1]], add=True)

    _wait(buf_a, gsem_a)


def _sc1a_body(feat_hbm, ns_hbm, labtab_hbm, dst_hbm,
               fs_hbm, lab_hbm, deg_hbm,
               ns_v, dst_v, buf_a, lab_v, acc2,
               gsem_a, gsem_b):
    c = lax.axis_index("c")
    s = lax.axis_index("s")
    wid = c * _NS + s

    _zero_acc(buf_a, acc2, s, _A1T)

    base = wid * _RT1
    pltpu.sync_copy(ns_hbm.at[pl.ds(base, _RT1)], ns_v)

    d0 = pltpu.async_copy(feat_hbm.at[ns_v.at[pl.ds(0, 128)]], buf_a, gsem_a)
    d0.wait()
    pltpu.sync_copy(buf_a, fs_hbm.at[pl.ds(base, 128)])
    d1 = pltpu.async_copy(feat_hbm.at[ns_v.at[pl.ds(128, 128)]], buf_a,
                          gsem_b)
    d1.wait()
    pltpu.sync_copy(buf_a, fs_hbm.at[pl.ds(base + 128, 128)])
    d2 = pltpu.async_copy(feat_hbm.at[ns_v.at[pl.ds(256, 64)]],
                          buf_a.at[pl.ds(0, 64)], gsem_a)
    d2.wait()
    pltpu.sync_copy(buf_a.at[pl.ds(0, 64)], fs_hbm.at[pl.ds(base + 256, 64)])

    l0 = pltpu.async_copy(labtab_hbm.at[ns_v.at[pl.ds(0, 128)]],
                          lab_v.at[pl.ds(0, 128)], gsem_a)
    l1 = pltpu.async_copy(labtab_hbm.at[ns_v.at[pl.ds(128, 128)]],
                          lab_v.at[pl.ds(128, 128)], gsem_b)
    l0.wait()
    l1.wait()
    l2 = pltpu.async_copy(labtab_hbm.at[ns_v.at[pl.ds(256, 64)]],
                          lab_v.at[pl.ds(256, 64)], gsem_a)
    l2.wait()
    pltpu.sync_copy(lab_v, lab_hbm.at[pl.ds(base, _RT1)])

    @pl.loop(0, _B)
    def _o(r):
        for k in range(_D // 16):
            buf_a[r, pl.ds(k * 16, 16)] = jnp.ones((16,), F32)

    plsc.subcore_barrier()

    erow = wid * _EB1
    for ch in range(_EB1 // _CB):
        pltpu.sync_copy(dst_hbm.at[pl.ds(erow + ch * _CB, _CB)], dst_v)

        @pl.loop(0, _CB)
        def _deg(j):
            pltpu.sync_copy(buf_a, acc2.at[dst_v.at[j]], add=True)

    plsc.subcore_barrier()
    pltpu.sync_copy(acc2.at[pl.ds(s * _A1T, _A1T)],
                    deg_hbm.at[pl.ds(c * _NSP + s * _A1T, _A1T)])


_sc1a = functools.partial(
    pl.kernel,
    out_type=[
        jax.ShapeDtypeStruct((_NSP, _D), F32),
        jax.ShapeDtypeStruct((_NSP,), I32),
        jax.ShapeDtypeStruct((2 * _NSP, _D), F32),
    ],
    mesh=_mesh,
    scratch_types=[
        pltpu.VMEM((_RT1,), I32),
        pltpu.VMEM((_CB, _B), I32),
        pltpu.VMEM((_B, _D), F32),
        pltpu.VMEM((_RT1,), I32),
        pltpu.VMEM_SHARED((_A1R, _D), F32),
        pltpu.SemaphoreType.DMA,
        pltpu.SemaphoreType.DMA,
    ],
)(_sc1a_body)


def _sc1b_body(fs_hbm, src_hbm, dst_hbm,
               agg1_hbm,
               idx_v, dst_v, buf_a, buf_b, acc,
               gsem_a, gsem_b):
    c = lax.axis_index("c")
    s = lax.axis_index("s")
    wid = c * _NS + s

    _zero_acc(buf_a, acc, s, _A1T)
    plsc.subcore_barrier()

    erow = wid * _EB1
    for ch in range(_EB1 // _CB):
        r0 = erow + ch * _CB
        pltpu.sync_copy(src_hbm.at[pl.ds(r0, _CB)], idx_v)
        pltpu.sync_copy(dst_hbm.at[pl.ds(r0, _CB)], dst_v)
        _agg_chunk(fs_hbm, idx_v, dst_v, buf_a, buf_b, acc, gsem_a, gsem_b)

    plsc.subcore_barrier()
    pltpu.sync_copy(acc.at[pl.ds(s * _A1T, _A1T)],
                    agg1_hbm.at[pl.ds(c * _NSP + s * _A1T, _A1T)])


_sc1b = functools.partial(
    pl.kernel,
    out_type=jax.ShapeDtypeStruct((2 * _NSP, _D), F32),
    mesh=_mesh,
    scratch_types=[
        pltpu.VMEM((_CB, _B), I32),
        pltpu.VMEM((_CB, _B), I32),
        pltpu.VMEM((_B, _D), F32),
        pltpu.VMEM((_B, _D), F32),
        pltpu.VMEM_SHARED((_A1R, _D), F32),
        pltpu.SemaphoreType.DMA,
        pltpu.SemaphoreType.DMA,
    ],
)(_sc1b_body)


def _sc2_body(y2a_hbm, y2b_hbm, src_hbm, dst_hbm,
              agg2_hbm,
              idx_v, dst_v, buf_a, buf_b, acc, gsem_a, gsem_b):
    c = lax.axis_index("c")
    s = lax.axis_index("s")

    _zero_acc(buf_a, acc, s, _ZT)
    plsc.subcore_barrier()

    erow = s * _EB2
    for ch in range(_EB2 // _CB):
        r0 = erow + ch * _CB
        pltpu.sync_copy(src_hbm.at[pl.ds(r0, _CB)], idx_v)
        pltpu.sync_copy(dst_hbm.at[pl.ds(r0, _CB)], dst_v)

        @pl.when(c == 0)
        def _():
            _agg_chunk(y2a_hbm, idx_v, dst_v, buf_a, buf_b, acc,
                       gsem_a, gsem_b)

        @pl.when(c == 1)
        def _():
            _agg_chunk(y2b_hbm, idx_v, dst_v, buf_a, buf_b, acc,
                       gsem_a, gsem_b)

    plsc.subcore_barrier()
    pltpu.sync_copy(acc.at[pl.ds(s * _ZT, _ZT)],
                    agg2_hbm.at[pl.ds(c * _NSP + s * _ZT, _ZT)])


_sc2 = functools.partial(
    pl.kernel,
    out_type=jax.ShapeDtypeStruct((2 * _NSP, _D), F32),
    mesh=_mesh,
    scratch_types=[
        pltpu.VMEM((_CB, _B), I32),
        pltpu.VMEM((_CB, _B), I32),
        pltpu.VMEM((_B, _D), F32),
        pltpu.VMEM((_B, _D), F32),
        pltpu.VMEM_SHARED((_NSP, _D), F32),
        pltpu.SemaphoreType.DMA,
        pltpu.SemaphoreType.DMA,
    ],
)(_sc2_body)


_R = 512
_G = _NSP // _R


def _tc1_body(fs, agg, dgp, ss, w1s, b1s, sn, w1n, b1n, w2s, b2s, w2n,
              xs2_o, y2a_o, y2b_o):
    dg = dgp[0, :, :1] + dgp[1, :, :1]
    scale = 1.0 / jnp.maximum(dg, 1.0)
    w1se = jnp.dot(ss[...], w1s[...], preferred_element_type=F32)
    w1ne = jnp.dot(sn[...], w1n[...], preferred_element_type=F32)
    a1 = (agg[0] + agg[1]) * scale
    xs = jnp.dot(fs[...], w1se, preferred_element_type=F32) + b1s[...]
    xn = jnp.dot(a1, w1ne, preferred_element_type=F32) + b1n[...]
    hs = jnp.maximum(xs, 0.0)
    hn = jnp.maximum(xn, 0.0)
    w2s_ = w2s[...]
    w2n_ = w2n[...]
    xs2_o[...] = (jnp.dot(hs, w2s_[:_H], preferred_element_type=F32)
                  + jnp.dot(hn, w2s_[_H:], preferred_element_type=F32) + b2s[...])
    y2 = (jnp.dot(hs, w2n_[:_H], preferred_element_type=F32)
          + jnp.dot(hn, w2n_[_H:], preferred_element_type=F32))
    y2a_o[...] = y2[:, :_D]
    y2b_o[...] = y2[:, _D:]


def _tc1(fs, agg, dgp, ss, w1s, b1s, sn, w1n, b1n, w2s, b2s, w2n):
    full = lambda shp: pl.BlockSpec(shp, lambda i: tuple(0 for _ in shp))
    return pl.pallas_call(
        _tc1_body,
        grid=(_G,),
        in_specs=[
            pl.BlockSpec((_R, _D), lambda i: (i, 0)),
            pl.BlockSpec((2, _R, _D), lambda i: (0, i, 0)),
            pl.BlockSpec((2, _R, _D), lambda i: (0, i, 0)),
            full((_D, _D)), full((_D, _H)), full((1, _H)),
            full((_D, _D)), full((_D, _H)), full((1, _H)),
            full((2 * _H, _H)), full((1, _H)), full((2 * _H, _H)),
        ],
        out_specs=[
            pl.BlockSpec((_R, _H), lambda i: (i, 0)),
            pl.BlockSpec((_R, _D), lambda i: (i, 0)),
            pl.BlockSpec((_R, _D), lambda i: (i, 0)),
        ],
        out_shape=[
            jax.ShapeDtypeStruct((_NSP, _H), F32),
            jax.ShapeDtypeStruct((_NSP, _D), F32),
            jax.ShapeDtypeStruct((_NSP, _D), F32),
        ],
    )(fs, agg, dgp, ss, w1s, b1s, sn, w1n, b1n, w2s, b2s, w2n)


def _tc2_body(xs2, agg, dgp, b2n, wc, bc, labf, pred_o, lab_o):
    dg = dgp[0, :, :1] + dgp[1, :, :1]
    scale = 1.0 / jnp.maximum(dg, 1.0)
    xn2 = jnp.concatenate([agg[0], agg[1]], axis=1) * scale + b2n[...]
    h2 = jnp.concatenate([jnp.maximum(xs2[...], 0.0), jnp.maximum(xn2, 0.0)],
                         axis=1)
    ssum = jnp.sum(h2 * h2, axis=1, keepdims=True)
    emb = h2 / jnp.maximum(jnp.sqrt(ssum), 1e-12)
    pred_o[...] = jnp.dot(emb, wc[...], preferred_element_type=F32) + bc[...]
    cls = lax.broadcasted_iota(I32, (_R, _D), 1).astype(F32)
    lab_o[...] = (cls == labf[...]).astype(F32)


def _tc2(xs2, agg, dgp, b2n, wc, bc, labf):
    full = lambda shp: pl.BlockSpec(shp, lambda i: tuple(0 for _ in shp))
    return pl.pallas_call(
        _tc2_body,
        grid=(_G,),
        in_specs=[
            pl.BlockSpec((_R, _H), lambda i: (i, 0)),
            pl.BlockSpec((2, _R, _D), lambda i: (0, i, 0)),
            pl.BlockSpec((2, _R, _D), lambda i: (0, i, 0)),
            full((1, _H)),
            full((2 * _H, _D)), full((1, _D)),
            pl.BlockSpec((_R, 1), lambda i: (i, 0)),
        ],
        out_specs=[
            pl.BlockSpec((_R, _D), lambda i: (i, 0)),
            pl.BlockSpec((_R, _D), lambda i: (i, 0)),
        ],
        out_shape=[
            jax.ShapeDtypeStruct((_NSP, _D), F32),
            jax.ShapeDtypeStruct((_NSP, _D), F32),
        ],
    )(xs2, agg, dgp, b2n, wc, bc, labf)


def kernel(node_subgraph, edge_index, edge_vals, feat_full, label_full,
           label_full_cat, mask_self, mask_neigh, W1_self, b1_self, W1_neigh,
           b1_neigh, W2_self, b2_self, W2_neigh, b2_neigh, Wc, bc):
    ns_pad = jnp.concatenate(
        [node_subgraph, jnp.zeros((_NSP - _N_SUB,), I32)])
    src_pad = jnp.concatenate(
        [edge_index[1], jnp.zeros((_EP - _E,), I32)]).reshape(_EPR, _B)
    dst_pad = jnp.concatenate(
        [edge_index[0], jnp.full((_EP - _E,), _N_SUB, I32)]).reshape(_EPR, _B)

    iota_d = jnp.arange(_D, dtype=I32)
    s_self = jnp.pad((iota_d[:, None] == mask_self[None, :]).astype(F32),
                     ((0, 0), (0, _D - mask_self.shape[0])))
    s_neigh = jnp.pad((iota_d[:, None] == mask_neigh[None, :]).astype(F32),
                      ((0, 0), (0, _D - mask_neigh.shape[0])))
    w1s_p = jnp.pad(W1_self, ((0, _D - W1_self.shape[0]), (0, 0)))
    w1n_p = jnp.pad(W1_neigh, ((0, _D - W1_neigh.shape[0]), (0, 0)))
    wc_p = jnp.pad(Wc, ((0, 0), (0, _D - _C)))
    bc_p = jnp.pad(bc, (0, _D - _C)).reshape(1, _D)

    fs, labcat, deg = _sc1a(feat_full, ns_pad, label_full_cat, dst_pad)
    agg1 = _sc1b(fs, src_pad, dst_pad)
    dgp = deg.reshape(2, _NSP, _D)

    xs2, y2a, y2b = _tc1(
        fs, agg1.reshape(2, _NSP, _D), dgp,
        s_self, w1s_p, b1_self.reshape(1, _H),
        s_neigh, w1n_p, b1_neigh.reshape(1, _H),
        W2_self, b2_self.reshape(1, _H), W2_neigh)

    agg2 = _sc2(y2a, y2b, src_pad, dst_pad)

    pred_p, lab_p = _tc2(
        xs2, agg2.reshape(2, _NSP, _D), dgp,
        b2_neigh.reshape(1, _H), wc_p, bc_p,
        labcat.astype(F32).reshape(_NSP, 1))

    return (pred_p[:_N_SUB, :_C], lab_p[:_N_SUB, :_C], labcat[:_N_SUB])
